# baseline (speedup 1.0000x reference)
import functools

import jax
import jax.numpy as jnp
from jax import lax
from jax.experimental import pallas as pl
from jax.experimental.pallas import tpu as pltpu
from jax.experimental.pallas import tpu_sc as plsc

_INFO = plsc.get_sparse_core_info()
_NC = _INFO.num_cores
_NS = _INFO.num_subcores
_NW = _NC * _NS


@functools.partial(jax.jit, static_argnums=())
def kernel(user_ids, item_ids, user_table, item_table):
    B = user_ids.shape[0]
    D = user_table.shape[1]
    assert B % (8 * _NW) == 0
    b_per_w = B // _NW

    mesh = plsc.VectorSubcoreMesh(core_axis_name="c", subcore_axis_name="s")

    @functools.partial(
        pl.kernel,
        mesh=mesh,
        compiler_params=pltpu.CompilerParams(use_tc_tiling_on_sc=False),
        out_type=(
            jax.ShapeDtypeStruct((B, D), jnp.float32),
            jax.ShapeDtypeStruct((B, D), jnp.float32),
        ),
        scratch_types=[
            pltpu.VMEM((b_per_w,), jnp.int32),
            pltpu.VMEM((b_per_w,), jnp.int32),
            pltpu.VMEM((b_per_w, D), jnp.float32),
            pltpu.VMEM((b_per_w, D), jnp.float32),
            pltpu.SemaphoreType.DMA,
            pltpu.SemaphoreType.DMA,
        ],
    )
    def _gather(uid_hbm, iid_hbm, utab_hbm, itab_hbm, uout_hbm, iout_hbm,
                uidx_v, iidx_v, urows_v, irows_v, usem, isem):
        wid = lax.axis_index("s") * _NC + lax.axis_index("c")
        base = wid * b_per_w
        pltpu.sync_copy(uid_hbm.at[pl.ds(base, b_per_w)], uidx_v)
        pltpu.sync_copy(iid_hbm.at[pl.ds(base, b_per_w)], iidx_v)
        ucp = pltpu.async_copy(utab_hbm.at[uidx_v], urows_v, usem)
        icp = pltpu.async_copy(itab_hbm.at[iidx_v], irows_v, isem)
        ucp.wait()
        pltpu.sync_copy(urows_v, uout_hbm.at[pl.ds(base, b_per_w)])
        icp.wait()
        pltpu.sync_copy(irows_v, iout_hbm.at[pl.ds(base, b_per_w)])

    return _gather(user_ids, item_ids, user_table, item_table)
